# baseline (speedup 1.0000x reference)
import functools

import jax
import jax.numpy as jnp
from jax.experimental import pallas as pl

DIM = 2048
NH = 16
DOWN = 512
UP = 2048
RHD = 32
VHD = 128
HID = 1408
NE = 8
TOPK = 2
EPS = 1e-5
S = 2048

QHD = UP // NH
D_QK = QHD + RHD

BQ = 256
BK = 256


def _rmsnorm(h, w):
    return h * jax.lax.rsqrt(jnp.mean(h * h, axis=-1, keepdims=True) + EPS) * w


def _rope(t, cs):
    t2 = t.reshape(t.shape[:-1] + (-1, 2))
    c = cs[None, :, None, :, 0]
    s = cs[None, :, None, :, 1]
    o0 = t2[..., 0] * c - t2[..., 1] * s
    o1 = t2[..., 0] * s + t2[..., 1] * c
    return jnp.stack([o0, o1], axis=-1).reshape(t.shape)


def _flash_attn_kernel(q_ref, k_ref, v_ref, o_ref):
    qi = pl.program_id(1)
    q = q_ref[0]
    scale = 1.0 / jnp.sqrt(jnp.float32(D_QK))
    row_ids = qi * BQ + jax.lax.broadcasted_iota(jnp.int32, (BQ, BK), 0)

    def body(j, carry):
        acc, m, l = carry
        k = k_ref[0, pl.ds(j * BK, BK), :]
        v = v_ref[0, pl.ds(j * BK, BK), :]
        s = jax.lax.dot_general(q, k, (((1,), (1,)), ((), ())),
                                preferred_element_type=jnp.float32) * scale
        col_ids = j * BK + jax.lax.broadcasted_iota(jnp.int32, (BQ, BK), 1)
        s = jnp.where(col_ids <= row_ids, s, -1e30)
        m_new = jnp.maximum(m, jnp.max(s, axis=-1, keepdims=True))
        p = jnp.exp(s - m_new)
        alpha = jnp.exp(m - m_new)
        l_new = l * alpha + jnp.sum(p, axis=-1, keepdims=True)
        acc_new = acc * alpha + jax.lax.dot_general(
            p, v, (((1,), (0,)), ((), ())), preferred_element_type=jnp.float32)
        return acc_new, m_new, l_new

    acc = jnp.zeros((BQ, VHD), jnp.float32)
    m0 = jnp.full((BQ, 1), -jnp.inf, jnp.float32)
    l0 = jnp.zeros((BQ, 1), jnp.float32)
    acc, m, l = jax.lax.fori_loop(0, qi + 1, body, (acc, m0, l0))
    o_ref[0] = acc / l


@functools.partial(jax.jit, static_argnums=())
def _flash_attn(xq, xk, v):
    return pl.pallas_call(
        _flash_attn_kernel,
        grid=(NH, S // BQ),
        in_specs=[
            pl.BlockSpec((1, BQ, D_QK), lambda h, qi: (h, qi, 0)),
            pl.BlockSpec((1, S, D_QK), lambda h, qi: (h, 0, 0)),
            pl.BlockSpec((1, S, VHD), lambda h, qi: (h, 0, 0)),
        ],
        out_specs=pl.BlockSpec((1, BQ, VHD), lambda h, qi: (h, qi, 0)),
        out_shape=jax.ShapeDtypeStruct((NH, S, VHD), jnp.float32),
    )(xq, xk, v)


def kernel(x, pos_cis, attn_norm_w, Wdkv, bdkv, Wuk, buk, Wuv, buv, Wdq, bdq,
           Wuq, buq, Wqr, bqr, Wkr, bkr, Wo, bo, ffn_norm_w, gate_w,
           ew1, ew2, ew3, sw1, sw2, sw3):
    b, s, _ = x.shape
    h = _rmsnorm(x, attn_norm_w)
    c_kv = h @ Wdkv.T + bdkv
    k_c = (c_kv @ Wuk.T + buk).reshape(b, s, NH, QHD).transpose(0, 2, 1, 3)
    v = (c_kv @ Wuv.T + buv).reshape(b, s, NH, VHD).transpose(0, 2, 1, 3)
    k_pe = (h @ Wkr.T + bkr).reshape(b, s, 1, RHD)
    c_q = h @ Wdq.T + bdq
    q_c = (c_q @ Wuq.T + buq).reshape(b, s, NH, QHD).transpose(0, 2, 1, 3)
    q_pe = (c_q @ Wqr.T + bqr).reshape(b, s, NH, RHD)
    q_r = _rope(q_pe, pos_cis).transpose(0, 2, 1, 3)
    k_r = _rope(k_pe, pos_cis).transpose(0, 2, 1, 3)
    k_r = jnp.broadcast_to(k_r, (b, NH, s, RHD))
    xq = jnp.concatenate([q_c, q_r], axis=-1)[0]
    xk = jnp.concatenate([k_c, k_r], axis=-1)[0]
    o = _flash_attn(xq, xk, v[0])
    o = o.transpose(1, 0, 2).reshape(1, s, NH * VHD)
    h_att = x + (o @ Wo.T + bo)
    hn = _rmsnorm(h_att, ffn_norm_w)
    f = hn.reshape(-1, DIM)
    logits = f @ gate_w.T
    sc = jax.nn.softmax(logits, axis=-1)
    tw, ti = jax.lax.top_k(sc, TOPK)
    oh = jax.nn.one_hot(ti, NE, dtype=f.dtype)
    we = jnp.einsum('tk,tke->te', tw, oh)
    h1 = jnp.einsum('td,ehd->teh', f, ew1)
    h3 = jnp.einsum('td,ehd->teh', f, ew3)
    ye = jnp.einsum('teh,edh->ted', jax.nn.silu(h1) * h3, ew2)
    y = jnp.einsum('ted,te->td', ye, we)
    y = y + (jax.nn.silu(f @ sw1.T) * (f @ sw3.T)) @ sw2.T
    return h_att + y.reshape(b, s, DIM)


# speedup vs baseline: 1.1179x; 1.1179x over previous
import functools

import jax
import jax.numpy as jnp
from jax.experimental import pallas as pl
from jax.experimental.pallas import tpu as pltpu

DIM = 2048
NH = 16
DOWN = 512
UP = 2048
RHD = 32
VHD = 128
HID = 1408
NE = 8
TOPK = 2
EPS = 1e-5
S = 2048

QHD = UP // NH

BQ = 256
BK = 256
BM = 128
NHB = 2
BH = HID // NHB
P = S * TOPK + NE * BM
NBLK = P // BM
NTB = S // BM


def _rmsnorm(h, w):
    return h * jax.lax.rsqrt(jnp.mean(h * h, axis=-1, keepdims=True) + EPS) * w


def _rope(t, cs):
    t2 = t.reshape(t.shape[:-1] + (-1, 2))
    c = cs[:, None, :, 0]
    s = cs[:, None, :, 1]
    o0 = t2[..., 0] * c - t2[..., 1] * s
    o1 = t2[..., 0] * s + t2[..., 1] * c
    return jnp.stack([o0, o1], axis=-1).reshape(t.shape)


def _flash_kernel(qc_ref, qr_ref, kc_ref, kr_ref, v_ref, o_ref):
    qi = pl.program_id(1)
    qc = qc_ref[...]
    qr = qr_ref[0]
    scale = 1.0 / jnp.sqrt(jnp.float32(QHD + RHD))
    row_ids = qi * BQ + jax.lax.broadcasted_iota(jnp.int32, (BQ, BK), 0)

    def body(j, carry):
        acc, m, l = carry
        kc = kc_ref[pl.ds(j * BK, BK), :]
        kr = kr_ref[pl.ds(j * BK, BK), :]
        v = v_ref[pl.ds(j * BK, BK), :]
        s = jax.lax.dot_general(qc, kc, (((1,), (1,)), ((), ())),
                                preferred_element_type=jnp.float32)
        s += jax.lax.dot_general(qr, kr, (((1,), (1,)), ((), ())),
                                 preferred_element_type=jnp.float32)
        s *= scale
        col_ids = j * BK + jax.lax.broadcasted_iota(jnp.int32, (BQ, BK), 1)
        s = jnp.where(col_ids <= row_ids, s, -1e30)
        m_new = jnp.maximum(m, jnp.max(s, axis=-1, keepdims=True))
        p = jnp.exp(s - m_new)
        alpha = jnp.exp(m - m_new)
        l_new = l * alpha + jnp.sum(p, axis=-1, keepdims=True)
        acc_new = acc * alpha + jax.lax.dot_general(
            p, v, (((1,), (0,)), ((), ())), preferred_element_type=jnp.float32)
        return acc_new, m_new, l_new

    acc = jnp.zeros((BQ, VHD), jnp.float32)
    m0 = jnp.full((BQ, 1), -jnp.inf, jnp.float32)
    l0 = jnp.zeros((BQ, 1), jnp.float32)
    acc, m, l = jax.lax.fori_loop(0, qi + 1, body, (acc, m0, l0))
    o_ref[...] = acc / l


def _flash_attn(q_c, q_r, k_c, k_r, v):
    return pl.pallas_call(
        _flash_kernel,
        grid=(NH, S // BQ),
        in_specs=[
            pl.BlockSpec((BQ, QHD), lambda h, qi: (qi, h)),
            pl.BlockSpec((1, BQ, RHD), lambda h, qi: (h, qi, 0)),
            pl.BlockSpec((S, QHD), lambda h, qi: (0, h)),
            pl.BlockSpec((S, RHD), lambda h, qi: (0, 0)),
            pl.BlockSpec((S, VHD), lambda h, qi: (0, h)),
        ],
        out_specs=pl.BlockSpec((BQ, VHD), lambda h, qi: (qi, h)),
        out_shape=jax.ShapeDtypeStruct((S, NH * VHD), jnp.float32),
    )(q_c, q_r, k_c, k_r, v)


def _ffn_up_kernel(be_ref, fs_ref, w1_ref, w3_ref, g_ref):
    fs = fs_ref[...]
    h1 = jax.lax.dot_general(fs, w1_ref[0], (((1,), (1,)), ((), ())),
                             preferred_element_type=jnp.float32)
    h3 = jax.lax.dot_general(fs, w3_ref[0], (((1,), (1,)), ((), ())),
                             preferred_element_type=jnp.float32)
    g_ref[...] = h1 * jax.lax.logistic(h1) * h3


def _ffn_down_kernel(be_ref, g_ref, w2_ref, o_ref):
    o_ref[...] = jax.lax.dot_general(g_ref[...], w2_ref[0],
                                     (((1,), (1,)), ((), ())),
                                     preferred_element_type=jnp.float32)


def _grouped_ffn(be, fs, w1, w3, w2, nblk):
    g = pl.pallas_call(
        _ffn_up_kernel,
        grid_spec=pltpu.PrefetchScalarGridSpec(
            num_scalar_prefetch=1,
            grid=(nblk,),
            in_specs=[
                pl.BlockSpec((BM, DIM), lambda i, be: (i, 0)),
                pl.BlockSpec((1, HID, DIM), lambda i, be: (be[i], 0, 0)),
                pl.BlockSpec((1, HID, DIM), lambda i, be: (be[i], 0, 0)),
            ],
            out_specs=pl.BlockSpec((BM, HID), lambda i, be: (i, 0)),
        ),
        out_shape=jax.ShapeDtypeStruct((nblk * BM, HID), jnp.float32),
    )(be, fs, w1, w3)
    return pl.pallas_call(
        _ffn_down_kernel,
        grid_spec=pltpu.PrefetchScalarGridSpec(
            num_scalar_prefetch=1,
            grid=(nblk,),
            in_specs=[
                pl.BlockSpec((BM, HID), lambda i, be: (i, 0)),
                pl.BlockSpec((1, DIM, HID), lambda i, be: (be[i], 0, 0)),
            ],
            out_specs=pl.BlockSpec((BM, DIM), lambda i, be: (i, 0)),
        ),
        out_shape=jax.ShapeDtypeStruct((nblk * BM, DIM), jnp.float32),
    )(be, g, w2)


def kernel(x, pos_cis, attn_norm_w, Wdkv, bdkv, Wuk, buk, Wuv, buv, Wdq, bdq,
           Wuq, buq, Wqr, bqr, Wkr, bkr, Wo, bo, ffn_norm_w, gate_w,
           ew1, ew2, ew3, sw1, sw2, sw3):
    b, s, _ = x.shape
    xf = x.reshape(s, DIM)
    h = _rmsnorm(xf, attn_norm_w)
    c_kv = h @ Wdkv.T + bdkv
    c_q = h @ Wdq.T + bdq
    k_c = c_kv @ Wuk.T + buk
    v = c_kv @ Wuv.T + buv
    q_c = c_q @ Wuq.T + buq
    q_pe = (c_q @ Wqr.T + bqr).reshape(s, NH, RHD)
    k_pe = (h @ Wkr.T + bkr).reshape(s, 1, RHD)
    q_r = _rope(q_pe, pos_cis).transpose(1, 0, 2)
    k_r = _rope(k_pe, pos_cis).reshape(s, RHD)

    o = _flash_attn(q_c, q_r, k_c, k_r, v)
    h_att = xf + o @ Wo.T + bo

    f = _rmsnorm(h_att, ffn_norm_w)
    logits = f @ gate_w.T
    sc = jax.nn.softmax(logits, axis=-1)
    tw, ti = jax.lax.top_k(sc, TOPK)

    e_flat = ti.reshape(-1).astype(jnp.int32)
    order = jnp.argsort(e_flat, stable=True).astype(jnp.int32)
    sorted_e = e_flat[order]
    counts = jnp.sum(jax.nn.one_hot(e_flat, NE, dtype=jnp.int32), axis=0)
    pc = ((counts + BM - 1) // BM) * BM
    group_start = jnp.cumsum(counts) - counts
    padded_start = jnp.cumsum(pc) - pc
    r_in_group = jnp.arange(S * TOPK, dtype=jnp.int32) - group_start[sorted_e]
    dest = padded_start[sorted_e] + r_in_group
    src = jnp.zeros((P,), jnp.int32).at[dest].set(order)
    tok = src // TOPK
    blk_start = padded_start // BM
    bids = jnp.arange(NBLK, dtype=jnp.int32)
    blk_expert = (jnp.sum(bids[:, None] >= blk_start[None, :], axis=1)
                  .astype(jnp.int32) - 1)

    fs = jnp.take(f, tok, axis=0)
    ye = _grouped_ffn(blk_expert, fs, ew1, ew3, ew2, NBLK)
    y_sh = _grouped_ffn(jnp.zeros((NTB,), jnp.int32), f,
                        sw1[None], sw3[None], sw2[None], NTB)

    pos_flat = jnp.zeros((S * TOPK,), jnp.int32).at[order].set(dest)
    pos = pos_flat.reshape(S, TOPK)
    y = (tw[:, 0:1] * jnp.take(ye, pos[:, 0], axis=0)
         + tw[:, 1:2] * jnp.take(ye, pos[:, 1], axis=0)
         + y_sh)
    return (h_att + y).reshape(b, s, DIM)


# speedup vs baseline: 1.3262x; 1.1864x over previous
import functools

import jax
import jax.numpy as jnp
from jax.experimental import pallas as pl
from jax.experimental.pallas import tpu as pltpu

DIM = 2048
NH = 16
DOWN = 512
UP = 2048
RHD = 32
VHD = 128
HID = 1408
NE = 8
TOPK = 2
EPS = 1e-5
S = 2048

QHD = UP // NH

BQ = 512
BK = 512
BM = 128
NHB = 2
BH = HID // NHB
P = S * TOPK + NE * BM
NBLK = P // BM
NTB = S // BM


def _rmsnorm(h, w):
    return h * jax.lax.rsqrt(jnp.mean(h * h, axis=-1, keepdims=True) + EPS) * w


def _rope(t, cs):
    t2 = t.reshape(t.shape[:-1] + (-1, 2))
    c = cs[:, None, :, 0]
    s = cs[:, None, :, 1]
    o0 = t2[..., 0] * c - t2[..., 1] * s
    o1 = t2[..., 0] * s + t2[..., 1] * c
    return jnp.stack([o0, o1], axis=-1).reshape(t.shape)


def _flash_kernel(qc_ref, qr_ref, kc_ref, kr_ref, v_ref, o_ref):
    qi = pl.program_id(1)
    scale = 1.0 / jnp.sqrt(jnp.float32(QHD + RHD))
    qc = qc_ref[...] * scale
    qr = qr_ref[0] * scale

    def scores(j):
        kc = kc_ref[pl.ds(j * BK, BK), :]
        kr = kr_ref[pl.ds(j * BK, BK), :]
        s = jax.lax.dot_general(qc, kc, (((1,), (1,)), ((), ())),
                                preferred_element_type=jnp.float32)
        s += jax.lax.dot_general(qr, kr, (((1,), (1,)), ((), ())),
                                 preferred_element_type=jnp.float32)
        return s

    def update(j, s, carry):
        acc, m, l = carry
        v = v_ref[pl.ds(j * BK, BK), :]
        m_new = jnp.maximum(m, jnp.max(s, axis=-1, keepdims=True))
        p = jnp.exp(s - m_new)
        alpha = jnp.exp(m - m_new)
        l_new = l * alpha + jnp.sum(p, axis=-1, keepdims=True)
        acc_new = acc * alpha + jax.lax.dot_general(
            p, v, (((1,), (0,)), ((), ())), preferred_element_type=jnp.float32)
        return acc_new, m_new, l_new

    def body(j, carry):
        return update(j, scores(j), carry)

    acc = jnp.zeros((BQ, VHD), jnp.float32)
    m0 = jnp.full((BQ, 1), -jnp.inf, jnp.float32)
    l0 = jnp.zeros((BQ, 1), jnp.float32)
    carry = jax.lax.fori_loop(0, qi, body, (acc, m0, l0))
    s = scores(qi)
    mask = (jax.lax.broadcasted_iota(jnp.int32, (BQ, BK), 0)
            >= jax.lax.broadcasted_iota(jnp.int32, (BQ, BK), 1))
    s = jnp.where(mask, s, -1e30)
    acc, m, l = update(qi, s, carry)
    o_ref[...] = acc / l


def _flash_attn(q_c, q_r, k_c, k_r, v):
    return pl.pallas_call(
        _flash_kernel,
        grid=(NH, S // BQ),
        in_specs=[
            pl.BlockSpec((BQ, QHD), lambda h, qi: (qi, h)),
            pl.BlockSpec((1, BQ, RHD), lambda h, qi: (h, qi, 0)),
            pl.BlockSpec((S, QHD), lambda h, qi: (0, h)),
            pl.BlockSpec((S, RHD), lambda h, qi: (0, 0)),
            pl.BlockSpec((S, VHD), lambda h, qi: (0, h)),
        ],
        out_specs=pl.BlockSpec((BQ, VHD), lambda h, qi: (qi, h)),
        out_shape=jax.ShapeDtypeStruct((S, NH * VHD), jnp.float32),
    )(q_c, q_r, k_c, k_r, v)


def _ffn_up_kernel(be_ref, fs_ref, w1_ref, w3_ref, g_ref):
    fs = fs_ref[...]
    h1 = jax.lax.dot_general(fs, w1_ref[0], (((1,), (1,)), ((), ())),
                             preferred_element_type=jnp.float32)
    h3 = jax.lax.dot_general(fs, w3_ref[0], (((1,), (1,)), ((), ())),
                             preferred_element_type=jnp.float32)
    g_ref[...] = h1 * jax.lax.logistic(h1) * h3


def _ffn_down_kernel(be_ref, g_ref, w2_ref, o_ref):
    o_ref[...] = jax.lax.dot_general(g_ref[...], w2_ref[0],
                                     (((1,), (1,)), ((), ())),
                                     preferred_element_type=jnp.float32)


def _grouped_ffn(be, fs, w1, w3, w2, nblk):
    g = pl.pallas_call(
        _ffn_up_kernel,
        grid_spec=pltpu.PrefetchScalarGridSpec(
            num_scalar_prefetch=1,
            grid=(nblk,),
            in_specs=[
                pl.BlockSpec((BM, DIM), lambda i, be: (i, 0)),
                pl.BlockSpec((1, HID, DIM), lambda i, be: (be[i], 0, 0)),
                pl.BlockSpec((1, HID, DIM), lambda i, be: (be[i], 0, 0)),
            ],
            out_specs=pl.BlockSpec((BM, HID), lambda i, be: (i, 0)),
        ),
        out_shape=jax.ShapeDtypeStruct((nblk * BM, HID), jnp.float32),
    )(be, fs, w1, w3)
    return pl.pallas_call(
        _ffn_down_kernel,
        grid_spec=pltpu.PrefetchScalarGridSpec(
            num_scalar_prefetch=1,
            grid=(nblk,),
            in_specs=[
                pl.BlockSpec((BM, HID), lambda i, be: (i, 0)),
                pl.BlockSpec((1, DIM, HID), lambda i, be: (be[i], 0, 0)),
            ],
            out_specs=pl.BlockSpec((BM, DIM), lambda i, be: (i, 0)),
        ),
        out_shape=jax.ShapeDtypeStruct((nblk * BM, DIM), jnp.float32),
    )(be, g, w2)


def kernel(x, pos_cis, attn_norm_w, Wdkv, bdkv, Wuk, buk, Wuv, buv, Wdq, bdq,
           Wuq, buq, Wqr, bqr, Wkr, bkr, Wo, bo, ffn_norm_w, gate_w,
           ew1, ew2, ew3, sw1, sw2, sw3):
    b, s, _ = x.shape
    xf = x.reshape(s, DIM)
    h = _rmsnorm(xf, attn_norm_w)
    c_kv = h @ Wdkv.T + bdkv
    c_q = h @ Wdq.T + bdq
    k_c = c_kv @ Wuk.T + buk
    v = c_kv @ Wuv.T + buv
    q_c = c_q @ Wuq.T + buq
    q_pe = (c_q @ Wqr.T + bqr).reshape(s, NH, RHD)
    k_pe = (h @ Wkr.T + bkr).reshape(s, 1, RHD)
    q_r = _rope(q_pe, pos_cis).transpose(1, 0, 2)
    k_r = _rope(k_pe, pos_cis).reshape(s, RHD)

    o = _flash_attn(q_c, q_r, k_c, k_r, v)
    h_att = xf + o @ Wo.T + bo

    f = _rmsnorm(h_att, ffn_norm_w)
    logits = f @ gate_w.T
    sc = jax.nn.softmax(logits, axis=-1)
    tw, ti = jax.lax.top_k(sc, TOPK)

    e_flat = ti.reshape(-1).astype(jnp.int32)
    order = jnp.argsort(e_flat, stable=True).astype(jnp.int32)
    sorted_e = e_flat[order]
    counts = jnp.sum(jax.nn.one_hot(e_flat, NE, dtype=jnp.int32), axis=0)
    pc = ((counts + BM - 1) // BM) * BM
    group_start = jnp.cumsum(counts) - counts
    padded_start = jnp.cumsum(pc) - pc
    r_in_group = jnp.arange(S * TOPK, dtype=jnp.int32) - group_start[sorted_e]
    dest = padded_start[sorted_e] + r_in_group
    src = jnp.zeros((P,), jnp.int32).at[dest].set(order)
    tok = src // TOPK
    blk_start = padded_start // BM
    bids = jnp.arange(NBLK, dtype=jnp.int32)
    blk_expert = (jnp.sum(bids[:, None] >= blk_start[None, :], axis=1)
                  .astype(jnp.int32) - 1)

    fs = jnp.take(f, tok, axis=0)
    ye = _grouped_ffn(blk_expert, fs, ew1, ew3, ew2, NBLK)
    y_sh = _grouped_ffn(jnp.zeros((NTB,), jnp.int32), f,
                        sw1[None], sw3[None], sw2[None], NTB)

    pos_flat = jnp.zeros((S * TOPK,), jnp.int32).at[order].set(dest)
    pos = pos_flat.reshape(S, TOPK)
    y = (tw[:, 0:1] * jnp.take(ye, pos[:, 0], axis=0)
         + tw[:, 1:2] * jnp.take(ye, pos[:, 1], axis=0)
         + y_sh)
    return (h_att + y).reshape(b, s, DIM)
